# baseline (speedup 1.0000x reference)
import functools

import jax
import jax.numpy as jnp
from jax import lax
from jax.experimental import pallas as pl
from jax.experimental.pallas import tpu as pltpu
from jax.experimental.pallas import tpu_sc as plsc


def _emb_sum_call(table_0, table_1, ids0_p, ids1_p, B, L):
    H = table_0.shape[1]
    Ch = L // 2
    info = plsc.get_sparse_core_info()
    ncores, nsub, nlanes = info.num_cores, info.num_subcores, info.num_lanes
    NW = ncores * nsub
    BPW = B // NW
    HV = H // nlanes

    mesh = plsc.VectorSubcoreMesh(core_axis_name="c", subcore_axis_name="s")

    @functools.partial(
        pl.kernel,
        mesh=mesh,
        out_type=jax.ShapeDtypeStruct((B, L, H), jnp.float32),
        scratch_types=[
            pltpu.VMEM((BPW * 64,), jnp.int32),
            pltpu.VMEM((2 * BPW * 32,), jnp.int32),
            pltpu.VMEM((2, L, H), jnp.float32),
            pltpu.VMEM((16, H), jnp.float32),
            pltpu.VMEM((32, H), jnp.float32),
            pltpu.SemaphoreType.DMA,
            pltpu.SemaphoreType.DMA,
            pltpu.SemaphoreType.DMA,
            pltpu.SemaphoreType.DMA,
            pltpu.SemaphoreType.DMA,
            pltpu.SemaphoreType.DMA,
        ],
    )
    def emb(t0, t1, i0, i1, out, idx0_v, idx1_v, buf_a, buf_t, buf_b,
            sem_a0, sem_a1, sem_t, sem_b, sem_s0, sem_s1):
        sems_a = (sem_a0, sem_a1)
        sems_s = (sem_s0, sem_s1)
        wid = lax.axis_index("s") * ncores + lax.axis_index("c")
        base = wid * BPW
        pltpu.sync_copy(i0.at[pl.ds(base * 64, BPW * 64)], idx0_v)
        for h in range(2):
            pltpu.sync_copy(i1.at[pl.ds((h * B + base) * 32, BPW * 32)],
                            idx1_v.at[pl.ds(h * BPW * 32, BPW * 32)])

        def issue_ga(it, slot):
            for k in range(3):
                iv = idx0_v[pl.ds(it * 64 + k * 16, 16)]
                pltpu.async_copy(t0.at[iv],
                                 buf_a.at[slot, pl.ds(k * 16, 16)],
                                 sems_a[slot])

        def wait_ga(it, slot):
            for k in range(3):
                iv = idx0_v[pl.ds(it * 64 + k * 16, 16)]
                pltpu.make_async_copy(
                    t0.at[iv], buf_a.at[slot, pl.ds(k * 16, 16)],
                    sems_a[slot]).wait()

        def issue_gt(it):
            iv = idx0_v[pl.ds(it * 64 + 48, 16)]
            pltpu.async_copy(t0.at[iv], buf_t, sem_t)

        def wait_gt(it):
            iv = idx0_v[pl.ds(it * 64 + 48, 16)]
            pltpu.make_async_copy(t0.at[iv], buf_t, sem_t).wait()

        def issue_gb(it, h):
            for k in range(2):
                iv = idx1_v[pl.ds((h * BPW + it) * 32 + k * 16, 16)]
                pltpu.async_copy(t1.at[iv],
                                 buf_b.at[pl.ds(k * 16, 16)], sem_b)

        def wait_gb(it, h):
            for k in range(2):
                iv = idx1_v[pl.ds((h * BPW + it) * 32 + k * 16, 16)]
                pltpu.make_async_copy(
                    t1.at[iv], buf_b.at[pl.ds(k * 16, 16)], sem_b).wait()

        issue_ga(0, 0)
        issue_ga(1, 1)
        issue_gt(0)
        issue_gb(0, 0)

        def add_half(slot, h):
            def addrow(r, c2):
                for cc in range(HV):
                    sl = pl.ds(cc * nlanes, nlanes)
                    ar = h * Ch + r
                    buf_a[slot, ar, sl] = buf_a[slot, ar, sl] + buf_b[r, sl]
                return c2
            lax.fori_loop(0, Ch, addrow, 0)

        def body(it, slot):
            wait_ga(it, slot)
            wait_gt(it)
            for rr in range(2):
                for cc in range(HV):
                    sl = pl.ds(cc * nlanes, nlanes)
                    buf_a[slot, 48 + rr, sl] = buf_t[rr, sl]

            @pl.when(it + 1 < BPW)
            def _():
                issue_gt(it + 1)
            wait_gb(it, 0)
            add_half(slot, 0)
            issue_gb(it, 1)
            @pl.when((it >= 1) & (it + 1 < BPW))
            def _():
                pltpu.make_async_copy(
                    buf_a.at[1 - slot], out.at[0], sems_s[1 - slot]).wait()
                issue_ga(it + 1, 1 - slot)
            wait_gb(it, 1)
            add_half(slot, 1)

            @pl.when(it + 1 < BPW)
            def _():
                issue_gb(it + 1, 0)
            pltpu.async_copy(buf_a.at[slot], out.at[base + it], sems_s[slot])

        def outer(it2, carry):
            body(it2 * 2, 0)
            body(it2 * 2 + 1, 1)
            return carry

        lax.fori_loop(0, BPW // 2, outer, 0)

        for slot in range(2):
            pltpu.make_async_copy(
                buf_a.at[slot], out.at[0], sems_s[slot]).wait()

    return emb(table_0, table_1, ids0_p, ids1_p)


def _prep_ids0(ids0):
    B, L = ids0.shape
    return jnp.pad(ids0, ((0, 0), (0, 64 - L))).reshape(-1)


def _prep_ids1(ids1):
    B, L = ids1.shape
    Ch = L // 2
    p = jnp.pad(ids1.reshape(B, 2, Ch), ((0, 0), (0, 0), (0, 7)))
    return p.transpose(1, 0, 2).reshape(-1)


def _pos_att_call(rel, typ, invf2, att_table):
    B, L = rel.shape
    T, Hh = att_table.shape
    BB = 64
    G = B // BB

    def body(rel_ref, typ_ref, invf_ref, tab_ref, out_ref):
        relv = rel_ref[...].astype(jnp.float32)
        x = relv[:, :, None] / invf_ref[...][None, None, :]
        col = lax.broadcasted_iota(jnp.int32, (BB, L, Hh), 2)
        pe = jnp.where(col < Hh // 2, jnp.sin(x), jnp.cos(x))
        typ3 = typ_ref[...][:, :, None]
        acc = pe
        for t in range(T):
            row = tab_ref[t, :][None, None, :]
            acc = acc + jnp.where(typ3 == t, row, 0.0)
        out_ref[...] = acc

    return pl.pallas_call(
        body,
        grid=(G,),
        in_specs=[
            pl.BlockSpec((BB, L), lambda i: (i, 0)),
            pl.BlockSpec((BB, L), lambda i: (i, 0)),
            pl.BlockSpec((Hh,), lambda i: (0,)),
            pl.BlockSpec((T, Hh), lambda i: (0, 0)),
        ],
        out_specs=pl.BlockSpec((BB, L, Hh), lambda i: (i, 0, 0)),
        out_shape=jax.ShapeDtypeStruct((B, L, Hh), jnp.float32),
    )(rel, typ, invf2, att_table)


def kernel(input_ids_0, input_ids_1, attention_type_ids,
           relative_position_ids, table_0, table_1, attn_type_table,
           inverse_freqs):
    B, L = input_ids_0.shape
    out1 = _emb_sum_call(table_0, table_1, _prep_ids0(input_ids_0),
                         _prep_ids1(input_ids_1), B, L)
    invf2 = jnp.concatenate([inverse_freqs, inverse_freqs])
    out2 = _pos_att_call(relative_position_ids, attention_type_ids,
                         invf2, attn_type_table)
    return (out1, out2)


# speedup vs baseline: 1.0021x; 1.0021x over previous
import functools

import jax
import jax.numpy as jnp
from jax import lax
from jax.experimental import pallas as pl
from jax.experimental.pallas import tpu as pltpu
from jax.experimental.pallas import tpu_sc as plsc


def _emb_sum_call(table_0, table_1, ids0_p, ids1_p, B, L):
    H = table_0.shape[1]
    Ch = L // 2
    info = plsc.get_sparse_core_info()
    ncores, nsub, nlanes = info.num_cores, info.num_subcores, info.num_lanes
    NW = ncores * nsub
    BPW = B // NW
    HV = H // nlanes

    mesh = plsc.VectorSubcoreMesh(core_axis_name="c", subcore_axis_name="s")

    @functools.partial(
        pl.kernel,
        mesh=mesh,
        out_type=jax.ShapeDtypeStruct((B, L, H), jnp.float32),
        scratch_types=[
            pltpu.VMEM((BPW * 64,), jnp.int32),
            pltpu.VMEM((2 * BPW * 32,), jnp.int32),
            pltpu.VMEM((2, L, H), jnp.float32),
            pltpu.VMEM((16, H), jnp.float32),
            pltpu.VMEM((32, H), jnp.float32),
            pltpu.SemaphoreType.DMA,
            pltpu.SemaphoreType.DMA,
            pltpu.SemaphoreType.DMA,
            pltpu.SemaphoreType.DMA,
            pltpu.SemaphoreType.DMA,
            pltpu.SemaphoreType.DMA,
        ],
    )
    def emb(t0, t1, i0, i1, out, idx0_v, idx1_v, buf_a, buf_t, buf_b,
            sem_a0, sem_a1, sem_t, sem_b, sem_s0, sem_s1):
        sems_a = (sem_a0, sem_a1)
        sems_s = (sem_s0, sem_s1)
        wid = lax.axis_index("s") * ncores + lax.axis_index("c")
        base = wid * BPW
        pltpu.sync_copy(i0.at[pl.ds(base * 64, BPW * 64)], idx0_v)
        for h in range(2):
            pltpu.sync_copy(i1.at[pl.ds((h * B + base) * 32, BPW * 32)],
                            idx1_v.at[pl.ds(h * BPW * 32, BPW * 32)])

        def issue_ga(it, slot):
            for k in range(3):
                iv = idx0_v[pl.ds(it * 64 + k * 16, 16)]
                pltpu.async_copy(t0.at[iv],
                                 buf_a.at[slot, pl.ds(k * 16, 16)],
                                 sems_a[slot])

        def wait_ga(it, slot):
            for k in range(3):
                pltpu.make_async_copy(
                    t0.at[pl.ds(0, 16)], buf_a.at[slot, pl.ds(k * 16, 16)],
                    sems_a[slot]).wait()

        def issue_gt(it):
            iv = idx0_v[pl.ds(it * 64 + 48, 16)]
            pltpu.async_copy(t0.at[iv], buf_t, sem_t)

        def wait_gt(it):
            pltpu.make_async_copy(t0.at[pl.ds(0, 16)], buf_t, sem_t).wait()

        def issue_gb(it, h):
            for k in range(2):
                iv = idx1_v[pl.ds((h * BPW + it) * 32 + k * 16, 16)]
                pltpu.async_copy(t1.at[iv],
                                 buf_b.at[pl.ds(k * 16, 16)], sem_b)

        def wait_gb(it, h):
            for k in range(2):
                pltpu.make_async_copy(
                    t1.at[pl.ds(0, 16)], buf_b.at[pl.ds(k * 16, 16)],
                    sem_b).wait()

        issue_ga(0, 0)
        issue_ga(1, 1)
        issue_gt(0)
        issue_gb(0, 0)

        def add_half(slot, h):
            def addrow(r, c2):
                for cc in range(HV):
                    sl = pl.ds(cc * nlanes, nlanes)
                    ar = h * Ch + r
                    buf_a[slot, ar, sl] = buf_a[slot, ar, sl] + buf_b[r, sl]
                return c2
            lax.fori_loop(0, Ch, addrow, 0)

        def body(it, slot):
            wait_ga(it, slot)
            wait_gt(it)
            for rr in range(2):
                for cc in range(HV):
                    sl = pl.ds(cc * nlanes, nlanes)
                    buf_a[slot, 48 + rr, sl] = buf_t[rr, sl]

            @pl.when(it + 1 < BPW)
            def _():
                issue_gt(it + 1)
            wait_gb(it, 0)
            add_half(slot, 0)
            issue_gb(it, 1)
            @pl.when((it >= 1) & (it + 1 < BPW))
            def _():
                pltpu.make_async_copy(
                    buf_a.at[1 - slot], out.at[0], sems_s[1 - slot]).wait()
                issue_ga(it + 1, 1 - slot)
            wait_gb(it, 1)
            add_half(slot, 1)

            @pl.when(it + 1 < BPW)
            def _():
                issue_gb(it + 1, 0)
            pltpu.async_copy(buf_a.at[slot], out.at[base + it], sems_s[slot])

        def outer(it2, carry):
            body(it2 * 2, 0)
            body(it2 * 2 + 1, 1)
            return carry

        lax.fori_loop(0, BPW // 2, outer, 0)

        for slot in range(2):
            pltpu.make_async_copy(
                buf_a.at[slot], out.at[0], sems_s[slot]).wait()

    return emb(table_0, table_1, ids0_p, ids1_p)


def _prep_ids0(ids0):
    B, L = ids0.shape
    return jnp.pad(ids0, ((0, 0), (0, 64 - L))).reshape(-1)


def _prep_ids1(ids1):
    B, L = ids1.shape
    Ch = L // 2
    p = jnp.pad(ids1.reshape(B, 2, Ch), ((0, 0), (0, 0), (0, 7)))
    return p.transpose(1, 0, 2).reshape(-1)


def _pos_att_call(rel, typ, invf2, att_table):
    B, L = rel.shape
    T, Hh = att_table.shape
    BB = 64
    G = B // BB

    def body(rel_ref, typ_ref, invf_ref, tab_ref, out_ref):
        relv = rel_ref[...].astype(jnp.float32)
        x = relv[:, :, None] / invf_ref[...][None, None, :]
        col = lax.broadcasted_iota(jnp.int32, (BB, L, Hh), 2)
        pe = jnp.where(col < Hh // 2, jnp.sin(x), jnp.cos(x))
        typ3 = typ_ref[...][:, :, None]
        acc = pe
        for t in range(T):
            row = tab_ref[t, :][None, None, :]
            acc = acc + jnp.where(typ3 == t, row, 0.0)
        out_ref[...] = acc

    return pl.pallas_call(
        body,
        grid=(G,),
        in_specs=[
            pl.BlockSpec((BB, L), lambda i: (i, 0)),
            pl.BlockSpec((BB, L), lambda i: (i, 0)),
            pl.BlockSpec((Hh,), lambda i: (0,)),
            pl.BlockSpec((T, Hh), lambda i: (0, 0)),
        ],
        out_specs=pl.BlockSpec((BB, L, Hh), lambda i: (i, 0, 0)),
        out_shape=jax.ShapeDtypeStruct((B, L, Hh), jnp.float32),
    )(rel, typ, invf2, att_table)


def kernel(input_ids_0, input_ids_1, attention_type_ids,
           relative_position_ids, table_0, table_1, attn_type_table,
           inverse_freqs):
    B, L = input_ids_0.shape
    out1 = _emb_sum_call(table_0, table_1, _prep_ids0(input_ids_0),
                         _prep_ids1(input_ids_1), B, L)
    invf2 = jnp.concatenate([inverse_freqs, inverse_freqs])
    out2 = _pos_att_call(relative_position_ids, attention_type_ids,
                         invf2, attn_type_table)
    return (out1, out2)


# speedup vs baseline: 2.6577x; 2.6522x over previous
import functools

import jax
import jax.numpy as jnp
from jax import lax
from jax.experimental import pallas as pl
from jax.experimental.pallas import tpu as pltpu
from jax.experimental.pallas import tpu_sc as plsc


def _emb_sum_call(table_0, table_1, ids0, ids1):
    H = table_0.shape[1]
    N = ids0.shape[0]
    info = plsc.get_sparse_core_info()
    ncores, nsub, nlanes = info.num_cores, info.num_subcores, info.num_lanes
    NW = ncores * nsub
    NPW = N // NW
    C = nlanes
    NCH = NPW // C
    NBUF = 2
    HV = H // nlanes

    mesh = plsc.VectorSubcoreMesh(core_axis_name="c", subcore_axis_name="s")

    @functools.partial(
        pl.kernel,
        mesh=mesh,
        out_type=jax.ShapeDtypeStruct((N, H), jnp.float32),
        scratch_types=[
            pltpu.VMEM((NPW,), jnp.int32),
            pltpu.VMEM((NPW,), jnp.int32),
            pltpu.VMEM((NBUF, C, H), jnp.float32),
            pltpu.VMEM((NBUF, C, H), jnp.float32),
            pltpu.VMEM((NBUF, C, H), jnp.float32),
            pltpu.SemaphoreType.DMA,
            pltpu.SemaphoreType.DMA,
            pltpu.SemaphoreType.DMA,
            pltpu.SemaphoreType.DMA,
            pltpu.SemaphoreType.DMA,
            pltpu.SemaphoreType.DMA,
        ],
    )
    def emb(t0, t1, i0, i1, out, idx0_v, idx1_v, buf_a, buf_b, obuf,
            sem_a0, sem_a1, sem_b0, sem_b1, sem_s0, sem_s1):
        sems_a = (sem_a0, sem_a1)
        sems_b = (sem_b0, sem_b1)
        sems_s = (sem_s0, sem_s1)
        wid = lax.axis_index("s") * ncores + lax.axis_index("c")
        base = wid * NPW
        pltpu.sync_copy(i0.at[pl.ds(base, NPW)], idx0_v)
        pltpu.sync_copy(i1.at[pl.ds(base, NPW)], idx1_v)

        def issue_gathers(i, b):
            iv0 = idx0_v[pl.ds(i * C, C)]
            iv1 = idx1_v[pl.ds(i * C, C)]
            pltpu.async_copy(t0.at[iv0], buf_a.at[b], sems_a[b])
            pltpu.async_copy(t1.at[iv1], buf_b.at[b], sems_b[b])

        for b in range(NBUF):
            issue_gathers(b, b)

        def outer(it, carry):
            g = it * NBUF
            for b in range(NBUF):
                i = g + b
                pltpu.make_async_copy(
                    t0.at[pl.ds(0, C)], buf_a.at[b], sems_a[b]).wait()
                pltpu.make_async_copy(
                    t1.at[pl.ds(0, C)], buf_b.at[b], sems_b[b]).wait()

                @pl.when(i >= NBUF)
                def _():
                    pltpu.make_async_copy(
                        obuf.at[b], out.at[pl.ds(0, C)], sems_s[b]).wait()

                def addrow(r, c2):
                    for cc in range(HV):
                        sl = pl.ds(cc * nlanes, nlanes)
                        obuf[b, r, sl] = buf_a[b, r, sl] + buf_b[b, r, sl]
                    return c2
                lax.fori_loop(0, C, addrow, 0)

                @pl.when(i + NBUF < NCH)
                def _():
                    issue_gathers(i + NBUF, b)

                pltpu.async_copy(
                    obuf.at[b], out.at[pl.ds(base + i * C, C)], sems_s[b])
            return carry

        lax.fori_loop(0, NCH // NBUF, outer, 0)

        for b in range(NBUF):
            pltpu.make_async_copy(
                obuf.at[b], out.at[pl.ds(0, C)], sems_s[b]).wait()

    return emb(table_0, table_1, ids0, ids1)


def _pos_att_call(rel, typ, invf2, att_table):
    B, L = rel.shape
    T, Hh = att_table.shape
    BB = 64
    G = B // BB

    def body(rel_ref, typ_ref, invf_ref, tab_ref, out_ref):
        relv = rel_ref[...].astype(jnp.float32)
        x = relv[:, :, None] / invf_ref[...][None, None, :]
        col = lax.broadcasted_iota(jnp.int32, (BB, L, Hh), 2)
        pe = jnp.where(col < Hh // 2, jnp.sin(x), jnp.cos(x))
        typ3 = typ_ref[...][:, :, None]
        acc = pe
        for t in range(T):
            row = tab_ref[t, :][None, None, :]
            acc = acc + jnp.where(typ3 == t, row, 0.0)
        out_ref[...] = acc

    return pl.pallas_call(
        body,
        grid=(G,),
        in_specs=[
            pl.BlockSpec((BB, L), lambda i: (i, 0)),
            pl.BlockSpec((BB, L), lambda i: (i, 0)),
            pl.BlockSpec((Hh,), lambda i: (0,)),
            pl.BlockSpec((T, Hh), lambda i: (0, 0)),
        ],
        out_specs=pl.BlockSpec((BB, L, Hh), lambda i: (i, 0, 0)),
        out_shape=jax.ShapeDtypeStruct((B, L, Hh), jnp.float32),
    )(rel, typ, invf2, att_table)


def kernel(input_ids_0, input_ids_1, attention_type_ids,
           relative_position_ids, table_0, table_1, attn_type_table,
           inverse_freqs):
    B, L = input_ids_0.shape
    H = table_0.shape[1]
    out1 = _emb_sum_call(
        table_0, table_1,
        input_ids_0.reshape(-1), input_ids_1.reshape(-1)).reshape(B, L, H)
    invf2 = jnp.concatenate([inverse_freqs, inverse_freqs])
    out2 = _pos_att_call(relative_position_ids, attention_type_ids,
                         invf2, attn_type_table)
    return (out1, out2)


# speedup vs baseline: 2.7899x; 1.0497x over previous
import functools

import jax
import jax.numpy as jnp
from jax import lax
from jax.experimental import pallas as pl
from jax.experimental.pallas import tpu as pltpu
from jax.experimental.pallas import tpu_sc as plsc


def _emb_sum_call(table_0, table_1, ids0, ids1):
    H = table_0.shape[1]
    N = ids0.shape[0]
    info = plsc.get_sparse_core_info()
    ncores, nsub, nlanes = info.num_cores, info.num_subcores, info.num_lanes
    NW = ncores * nsub
    NPW = N // NW
    C = nlanes
    NCH = NPW // C
    NBUF = 2
    HV = H // nlanes

    mesh = plsc.VectorSubcoreMesh(core_axis_name="c", subcore_axis_name="s")

    @functools.partial(
        pl.kernel,
        mesh=mesh,
        out_type=jax.ShapeDtypeStruct((N, H), jnp.float32),
        scratch_types=[
            pltpu.VMEM((NPW,), jnp.int32),
            pltpu.VMEM((NPW,), jnp.int32),
            pltpu.VMEM((NBUF, C, H), jnp.float32),
            pltpu.VMEM((NBUF, C, H), jnp.float32),
            pltpu.VMEM((NBUF, C, H), jnp.float32),
            pltpu.SemaphoreType.DMA,
            pltpu.SemaphoreType.DMA,
            pltpu.SemaphoreType.DMA,
            pltpu.SemaphoreType.DMA,
            pltpu.SemaphoreType.DMA,
            pltpu.SemaphoreType.DMA,
        ],
    )
    def emb(t0, t1, i0, i1, out, idx0_v, idx1_v, buf_a, buf_b, obuf,
            sem_a0, sem_a1, sem_b0, sem_b1, sem_s0, sem_s1):
        sems_a = (sem_a0, sem_a1)
        sems_b = (sem_b0, sem_b1)
        sems_s = (sem_s0, sem_s1)
        wid = lax.axis_index("s") * ncores + lax.axis_index("c")
        base = wid * NPW
        pltpu.sync_copy(i0.at[pl.ds(base, NPW)], idx0_v)
        pltpu.sync_copy(i1.at[pl.ds(base, NPW)], idx1_v)

        def issue_gathers(i, b):
            iv0 = idx0_v[pl.ds(i * C, C)]
            iv1 = idx1_v[pl.ds(i * C, C)]
            pltpu.async_copy(t0.at[iv0], buf_a.at[b], sems_a[b])
            pltpu.async_copy(t1.at[iv1], buf_b.at[b], sems_b[b])

        for b in range(NBUF):
            issue_gathers(b, b)

        def outer(it, carry):
            g = it * NBUF
            for b in range(NBUF):
                i = g + b
                pltpu.make_async_copy(
                    t0.at[pl.ds(0, C)], buf_a.at[b], sems_a[b]).wait()
                pltpu.make_async_copy(
                    t1.at[pl.ds(0, C)], buf_b.at[b], sems_b[b]).wait()

                @pl.when(i >= NBUF)
                def _():
                    pltpu.make_async_copy(
                        obuf.at[b], out.at[pl.ds(0, C)], sems_s[b]).wait()

                def addrow(r, c2):
                    for cc in range(HV):
                        sl = pl.ds(cc * nlanes, nlanes)
                        obuf[b, r, sl] = buf_a[b, r, sl] + buf_b[b, r, sl]
                    return c2
                lax.fori_loop(0, C, addrow, 0)

                @pl.when(i + NBUF < NCH)
                def _():
                    issue_gathers(i + NBUF, b)

                pltpu.async_copy(
                    obuf.at[b], out.at[pl.ds(base + i * C, C)], sems_s[b])
            return carry

        lax.fori_loop(0, NCH // NBUF, outer, 0)

        for b in range(NBUF):
            pltpu.make_async_copy(
                obuf.at[b], out.at[pl.ds(0, C)], sems_s[b]).wait()

    return emb(table_0, table_1, ids0, ids1)


def _pos_att_call(rel, typ, invf2, att_table):
    N = rel.shape[0]
    T, Hh = att_table.shape
    R = 512
    G = N // R

    def body(rel_ref, typ_ref, invf_ref, tab_ref, out_ref):
        relv = rel_ref[...].astype(jnp.float32)
        x = relv[:, None] / invf_ref[...][None, :]
        col = lax.broadcasted_iota(jnp.int32, (R, Hh), 1)
        pe = jnp.where(col < Hh // 2, jnp.sin(x), jnp.cos(x))
        typ2 = typ_ref[...][:, None]
        acc = pe
        for t in range(T):
            row = tab_ref[t, :][None, :]
            acc = acc + jnp.where(typ2 == t, row, 0.0)
        out_ref[...] = acc

    return pl.pallas_call(
        body,
        grid=(G,),
        in_specs=[
            pl.BlockSpec((R,), lambda i: (i,)),
            pl.BlockSpec((R,), lambda i: (i,)),
            pl.BlockSpec((Hh,), lambda i: (0,)),
            pl.BlockSpec((T, Hh), lambda i: (0, 0)),
        ],
        out_specs=pl.BlockSpec((R, Hh), lambda i: (i, 0)),
        out_shape=jax.ShapeDtypeStruct((N, Hh), jnp.float32),
    )(rel, typ, invf2, att_table)


def kernel(input_ids_0, input_ids_1, attention_type_ids,
           relative_position_ids, table_0, table_1, attn_type_table,
           inverse_freqs):
    B, L = input_ids_0.shape
    H = table_0.shape[1]
    Hh = attn_type_table.shape[1]
    out1 = _emb_sum_call(
        table_0, table_1,
        input_ids_0.reshape(-1), input_ids_1.reshape(-1)).reshape(B, L, H)
    invf2 = jnp.concatenate([inverse_freqs, inverse_freqs])
    out2 = _pos_att_call(
        relative_position_ids.reshape(-1), attention_type_ids.reshape(-1),
        invf2, attn_type_table).reshape(B, L, Hh)
    return (out1, out2)
